# baseline (speedup 1.0000x reference)
import functools

import jax
import jax.numpy as jnp
from jax import lax
from jax.experimental import pallas as pl
from jax.experimental.pallas import tpu as pltpu
from jax.experimental.pallas import tpu_sc as plsc

_RATIO = 0.75
_B, _T, _C = 256, 1024, 192
_R = int(_T * (1 - _RATIO))
_NC, _NS = 2, 16
_NW = _NC * _NS
_NROWS = _B * _R
_RPW = _NROWS // _NW
_K = 128
_NCH = _RPW // _K
_NBUF = 4
_LEAD = 2


def _gather_body(table, idxs, out, idx_v, bufs, gsem, psem):
    wid = lax.axis_index("s") * _NC + lax.axis_index("c")
    base = wid * _RPW
    pltpu.sync_copy(idxs.at[wid], idx_v)

    g = [None] * _NCH
    p = [None] * _NCH

    def start_gather(j):
        b = j % _NBUF
        g[j] = pltpu.async_copy(table.at[idx_v.at[j]], bufs.at[b], gsem.at[b])

    for j in range(min(_LEAD, _NCH)):
        start_gather(j)
    for j in range(_NCH):
        nxt = j + _LEAD
        if nxt < _NCH:
            prev = nxt - _NBUF
            if prev >= 0:
                p[prev].wait()
            start_gather(nxt)
        b = j % _NBUF
        g[j].wait()
        p[j] = pltpu.async_copy(
            bufs.at[b], out.at[pl.ds(base + j * _K, _K)], psem.at[b])
    for j in range(max(0, _NCH - _NBUF), _NCH):
        if p[j] is not None:
            p[j].wait()


_gather_rows = functools.partial(
    pl.kernel,
    out_type=jax.ShapeDtypeStruct((_NROWS, _C), jnp.float32),
    mesh=plsc.VectorSubcoreMesh(core_axis_name="c", subcore_axis_name="s"),
    scratch_types=[
        pltpu.VMEM((_NCH, _K), jnp.int32),
        pltpu.VMEM((_NBUF, _K, _C), jnp.float32),
        pltpu.SemaphoreType.DMA((_NBUF,)),
        pltpu.SemaphoreType.DMA((_NBUF,)),
    ],
    compiler_params=pltpu.CompilerParams(use_tc_tiling_on_sc=False),
)(_gather_body)


_CONSTS = []


def _perm_consts():
    if not _CONSTS:
        keys = jax.random.split(jax.random.key(42), _B)
        fwd = jax.vmap(lambda k: jax.random.permutation(k, _T))(keys).astype(
            jnp.int64)
        bwd = jnp.argsort(fwd, axis=1)
        row_base = (jnp.arange(_B, dtype=jnp.int32) * _T)[:, None]
        gidx = (fwd[:, :_R].astype(jnp.int32) + row_base).reshape(
            _NW, _NCH, _K)
        _CONSTS.append((fwd, bwd, gidx))
    return _CONSTS[0]


def kernel(patches):
    fwd, bwd, gidx = _perm_consts()
    table = patches.reshape(_B * _T, _C)
    out = _gather_rows(table, gidx)
    return (out.reshape(_B, _R, _C), fwd, bwd)
